# baseline (speedup 1.0000x reference)
import functools

import jax
import jax.numpy as jnp
from jax import lax
from jax.experimental import pallas as pl
from jax.experimental.pallas import tpu as pltpu
from jax.experimental.pallas import tpu_sc as plsc

N = 20000
N_PAD = 20480
NTILES = 16
CHUNK = N_PAD // NTILES
ITERS = CHUNK // 16
IOU_TR = 0.3
NEG_INF = float("-inf")

_mesh = plsc.VectorSubcoreMesh(core_axis_name="c", subcore_axis_name="s")


@functools.partial(
    pl.kernel,
    out_type=jax.ShapeDtypeStruct((32,), jnp.float32),
    mesh=_mesh,
    compiler_params=pltpu.CompilerParams(needs_layout_passes=False),
    scratch_types=(
        [pltpu.VMEM((CHUNK,), jnp.float32) for _ in range(8)]
        + [
            pltpu.VMEM((CHUNK,), jnp.float32),
            pltpu.VMEM((16,), jnp.float32),
            pltpu.VMEM((256,), jnp.float32),
            pltpu.VMEM((16,), jnp.float32),
            pltpu.SemaphoreType.DMA,
            pltpu.SemaphoreType.DMA,
            pltpu.VMEM_SHARED((512,), jnp.float32),
        ]
    ),
)
def _nms_sc(res_hbm, out_hbm,
            cb0, cb1, cb2, cb3, cb4, cb5, cb6, cb7, scvec, pub, lbuf, outv,
            sem_sc, sem_co, board_sp):
    cid = lax.axis_index("c")
    sid = lax.axis_index("s")

    lanes = lax.iota(jnp.int32, 16)
    neg_inf = jnp.float32(NEG_INF)
    cols = (cb0, cb1, cb2, cb3, cb4, cb5, cb6, cb7)

    copies = [
        pltpu.make_async_copy(
            res_hbm.at[pl.ds(k * N_PAD + sid * CHUNK, CHUNK)],
            v.at[pl.ds(0, CHUNK)], sem_sc if k < 2 else sem_co)
        for k, v in enumerate(cols)
    ]
    for c in copies:
        c.start()
    copies[0].wait()
    copies[1].wait()

    classf = cid.astype(jnp.float32)
    base_g = sid * CHUNK

    mval0 = jnp.full((16,), neg_inf, jnp.float32)
    midx0 = jnp.full((16,), base_g, jnp.int32)
    HALF = ITERS // 2

    def _upd(mval, midx, sc, rid):
        upd = sc > mval
        return jnp.where(upd, sc, mval), jnp.where(upd, rid + base_g, midx)

    def _merge(c1, c2):
        (m1, i1), (m2, i2) = c1, c2
        upd = m2 > m1
        return jnp.where(upd, m2, m1), jnp.where(upd, i2, i1)

    def _sc_at(rid):
        score = plsc.load_gather(cb0, [rid])
        clsv = plsc.load_gather(cb1, [rid])
        sc = jnp.where((clsv == classf) & (score >= 0.0), score, neg_inf)
        plsc.store_scatter(scvec, [rid], sc)
        return sc

    def pass_a(i, carry):
        m1, i1, m2, i2 = carry
        rid1 = i * 16 + lanes
        rid2 = (i + HALF) * 16 + lanes
        m1, i1 = _upd(m1, i1, _sc_at(rid1), rid1)
        m2, i2 = _upd(m2, i2, _sc_at(rid2), rid2)
        return m1, i1, m2, i2

    c_a = lax.fori_loop(0, HALF, pass_a, (mval0, midx0, mval0, midx0))
    mval_a, midx_a = _merge((c_a[0], c_a[1]), (c_a[2], c_a[3]))

    for c in copies[2:]:
        c.wait()

    def publish(r, mval, midx):
        m = jnp.max(mval)
        g = jnp.min(jnp.where(mval == m, midx, jnp.int32(2**30)))
        lidx = jnp.full((16,), g - base_g, jnp.int32)
        v = jnp.where(lanes == 0, m, g.astype(jnp.float32))
        for k in range(8):
            v = jnp.where(lanes == 2 + k,
                          plsc.load_gather(cols[k], [lidx]), v)
        pub[...] = v
        pltpu.sync_copy(pub, board_sp.at[pl.ds(r * 256 + sid * 16, 16)])

    def read_reduce(r):
        pltpu.sync_copy(board_sp.at[pl.ds(r * 256, 256)], lbuf)
        mcol = plsc.load_gather(lbuf, [lanes * 16])
        icol = plsc.load_gather(lbuf, [lanes * 16 + 1])
        mg = jnp.max(mcol)
        gidx = jnp.min(jnp.where(mcol == mg, icol, jnp.float32(1e30)))
        w = jnp.min(jnp.where((mcol == mg) & (icol == gidx), lanes,
                              jnp.int32(16)))
        return jnp.full((16,), w, jnp.int32)

    def out_row(wv, slot):
        orow = plsc.load_gather(lbuf, [wv * 16 + lanes + 2], mask=lanes < 8)
        plsc.store_scatter(outv, [lanes + slot * 8], orow, mask=lanes < 8)

    publish(0, mval_a, midx_a)
    plsc.subcore_barrier()
    wv = read_reduce(0)

    def coord(k):
        return plsc.load_gather(lbuf, [wv * 16 + 4 + k])

    bx1, by1, bz1 = coord(0), coord(1), coord(2)
    bx2, by2, bz2 = coord(3), coord(4), coord(5)
    v1 = (jnp.maximum(bx2 - bx1, 0.0) * jnp.maximum(by2 - by1, 0.0)
          * jnp.maximum(bz2 - bz1, 0.0))

    @pl.when(sid == 0)
    def _():
        out_row(wv, 0)

    tr = jnp.float32(IOU_TR)
    eps = jnp.float32(1e-7)

    def _supp_at(rid):
        sc = plsc.load_gather(scvec, [rid])
        x1 = plsc.load_gather(cb2, [rid])
        y1 = plsc.load_gather(cb3, [rid])
        z1 = plsc.load_gather(cb4, [rid])
        x2 = plsc.load_gather(cb5, [rid])
        y2 = plsc.load_gather(cb6, [rid])
        z2 = plsc.load_gather(cb7, [rid])
        wx = jnp.maximum(jnp.minimum(bx2, x2) - jnp.maximum(bx1, x1), 0.0)
        wy = jnp.maximum(jnp.minimum(by2, y2) - jnp.maximum(by1, y1), 0.0)
        wz = jnp.maximum(jnp.minimum(bz2, z2) - jnp.maximum(bz1, z1), 0.0)
        inter = wx * wy * wz
        v2 = (jnp.maximum(x2 - x1, 0.0) * jnp.maximum(y2 - y1, 0.0)
              * jnp.maximum(z2 - z1, 0.0))
        keep = inter <= tr * (v1 + v2 - inter + eps)
        return jnp.where(keep, sc, neg_inf)

    def pass_b(i, carry):
        m1, i1, m2, i2 = carry
        for s in range(2):
            rid1 = (2 * i + s) * 16 + lanes
            rid2 = (2 * i + s + HALF) * 16 + lanes
            m1, i1 = _upd(m1, i1, _supp_at(rid1), rid1)
            m2, i2 = _upd(m2, i2, _supp_at(rid2), rid2)
        return m1, i1, m2, i2

    c_b = lax.fori_loop(0, HALF // 2, pass_b, (mval0, midx0, mval0, midx0))
    mval_b, midx_b = _merge((c_b[0], c_b[1]), (c_b[2], c_b[3]))

    publish(1, mval_b, midx_b)
    plsc.subcore_barrier()

    @pl.when(sid == 0)
    def _():
        wv2 = read_reduce(1)
        out_row(wv2, 1)
        pltpu.sync_copy(outv, out_hbm.at[pl.ds(cid * 16, 16)])


def kernel(results):
    cols = jnp.pad(results.T, ((0, 0), (0, N_PAD - N)), constant_values=-1.0)
    out = _nms_sc(cols.reshape(-1))
    return out.reshape(4, 8)


# speedup vs baseline: 1.0144x; 1.0144x over previous
import functools

import jax
import jax.numpy as jnp
from jax import lax
from jax.experimental import pallas as pl
from jax.experimental.pallas import tpu as pltpu
from jax.experimental.pallas import tpu_sc as plsc

N = 20000
N_PAD = 20480
NTILES = 16
CHUNK = N_PAD // NTILES
ITERS = CHUNK // 16
IOU_TR = 0.3
NEG_INF = float("-inf")

_mesh = plsc.VectorSubcoreMesh(core_axis_name="c", subcore_axis_name="s")


@functools.partial(
    pl.kernel,
    out_type=jax.ShapeDtypeStruct((32,), jnp.float32),
    mesh=_mesh,
    compiler_params=pltpu.CompilerParams(needs_layout_passes=False),
    scratch_types=(
        [pltpu.VMEM((CHUNK,), jnp.float32) for _ in range(8)]
        + [
            pltpu.VMEM((CHUNK,), jnp.float32),
            pltpu.VMEM((16,), jnp.float32),
            pltpu.VMEM((256,), jnp.float32),
            pltpu.VMEM((16,), jnp.float32),
            pltpu.SemaphoreType.DMA,
            pltpu.SemaphoreType.DMA,
            pltpu.VMEM_SHARED((512,), jnp.float32),
        ]
    ),
)
def _nms_sc(res_hbm, out_hbm,
            cb0, cb1, cb2, cb3, cb4, cb5, cb6, cb7, scvec, pub, lbuf, outv,
            sem_sc, sem_co, board_sp):
    cid = lax.axis_index("c")
    sid = lax.axis_index("s")

    lanes = lax.iota(jnp.int32, 16)
    neg_inf = jnp.float32(NEG_INF)
    cols = (cb0, cb1, cb2, cb3, cb4, cb5, cb6, cb7)

    copies = [
        pltpu.make_async_copy(
            res_hbm.at[pl.ds(k * N_PAD + sid * CHUNK, CHUNK)],
            v.at[pl.ds(0, CHUNK)], sem_sc if k < 2 else sem_co)
        for k, v in enumerate(cols)
    ]
    for c in copies:
        c.start()
    copies[0].wait()
    copies[1].wait()

    classf = cid.astype(jnp.float32)
    base_g = sid * CHUNK

    mval0 = jnp.full((16,), neg_inf, jnp.float32)
    midx0 = jnp.full((16,), base_g, jnp.int32)
    HALF = ITERS // 2

    def _upd(mval, midx, sc, rid):
        upd = sc > mval
        return jnp.where(upd, sc, mval), jnp.where(upd, rid + base_g, midx)

    def _merge(c1, c2):
        (m1, i1), (m2, i2) = c1, c2
        upd = m2 > m1
        return jnp.where(upd, m2, m1), jnp.where(upd, i2, i1)

    def _sc_at(rid):
        score = plsc.load_gather(cb0, [rid])
        clsv = plsc.load_gather(cb1, [rid])
        sc = jnp.where((clsv == classf) & (score >= 0.0), score, neg_inf)
        plsc.store_scatter(scvec, [rid], sc)
        return sc

    def pass_a(i, carry):
        m1, i1, m2, i2 = carry
        rid1 = i * 16 + lanes
        rid2 = (i + HALF) * 16 + lanes
        m1, i1 = _upd(m1, i1, _sc_at(rid1), rid1)
        m2, i2 = _upd(m2, i2, _sc_at(rid2), rid2)
        return m1, i1, m2, i2

    c_a = lax.fori_loop(0, HALF, pass_a, (mval0, midx0, mval0, midx0))
    mval_a, midx_a = _merge((c_a[0], c_a[1]), (c_a[2], c_a[3]))

    for c in copies[2:]:
        c.wait()

    def publish(r, mval, midx):
        m = jnp.max(mval)
        g = jnp.min(jnp.where(mval == m, midx, jnp.int32(2**30)))
        lidx = jnp.full((16,), g - base_g, jnp.int32)
        v = jnp.where(lanes == 0, m, g.astype(jnp.float32))
        for k in range(8):
            v = jnp.where(lanes == 2 + k,
                          plsc.load_gather(cols[k], [lidx]), v)
        pub[...] = v
        pltpu.sync_copy(pub, board_sp.at[pl.ds(r * 256 + sid * 16, 16)])

    def read_reduce(r):
        pltpu.sync_copy(board_sp.at[pl.ds(r * 256, 256)], lbuf)
        mcol = plsc.load_gather(lbuf, [lanes * 16])
        icol = plsc.load_gather(lbuf, [lanes * 16 + 1])
        mg = jnp.max(mcol)
        gidx = jnp.min(jnp.where(mcol == mg, icol, jnp.float32(1e30)))
        w = jnp.min(jnp.where((mcol == mg) & (icol == gidx), lanes,
                              jnp.int32(16)))
        return jnp.full((16,), w, jnp.int32)

    def out_row(wv, slot):
        orow = plsc.load_gather(lbuf, [wv * 16 + lanes + 2], mask=lanes < 8)
        plsc.store_scatter(outv, [lanes + slot * 8], orow, mask=lanes < 8)

    publish(0, mval_a, midx_a)
    plsc.subcore_barrier()
    wv = read_reduce(0)

    def coord(k):
        return plsc.load_gather(lbuf, [wv * 16 + 4 + k])

    bx1, by1, bz1 = coord(0), coord(1), coord(2)
    bx2, by2, bz2 = coord(3), coord(4), coord(5)
    v1 = (jnp.maximum(bx2 - bx1, 0.0) * jnp.maximum(by2 - by1, 0.0)
          * jnp.maximum(bz2 - bz1, 0.0))

    @pl.when(sid == 0)
    def _():
        out_row(wv, 0)

    tr = jnp.float32(IOU_TR)
    eps = jnp.float32(1e-7)

    def _supp_at(rid):
        sc = plsc.load_gather(scvec, [rid])
        x1 = plsc.load_gather(cb2, [rid])
        y1 = plsc.load_gather(cb3, [rid])
        z1 = plsc.load_gather(cb4, [rid])
        x2 = plsc.load_gather(cb5, [rid])
        y2 = plsc.load_gather(cb6, [rid])
        z2 = plsc.load_gather(cb7, [rid])
        wx = jnp.maximum(jnp.minimum(bx2, x2) - jnp.maximum(bx1, x1), 0.0)
        wy = jnp.maximum(jnp.minimum(by2, y2) - jnp.maximum(by1, y1), 0.0)
        wz = jnp.maximum(jnp.minimum(bz2, z2) - jnp.maximum(bz1, z1), 0.0)
        inter = wx * wy * wz
        v2 = (jnp.maximum(x2 - x1, 0.0) * jnp.maximum(y2 - y1, 0.0)
              * jnp.maximum(z2 - z1, 0.0))
        keep = inter <= tr * (v1 + v2 - inter + eps)
        return jnp.where(keep, sc, neg_inf)

    def pass_b(i, carry):
        m1, i1, m2, i2 = carry
        rid1 = i * 16 + lanes
        rid2 = (i + HALF) * 16 + lanes
        m1, i1 = _upd(m1, i1, _supp_at(rid1), rid1)
        m2, i2 = _upd(m2, i2, _supp_at(rid2), rid2)
        return m1, i1, m2, i2

    c_b = lax.fori_loop(0, HALF, pass_b, (mval0, midx0, mval0, midx0))
    mval_b, midx_b = _merge((c_b[0], c_b[1]), (c_b[2], c_b[3]))

    publish(1, mval_b, midx_b)
    plsc.subcore_barrier()

    @pl.when(sid == 0)
    def _():
        wv2 = read_reduce(1)
        out_row(wv2, 1)
        pltpu.sync_copy(outv, out_hbm.at[pl.ds(cid * 16, 16)])


def kernel(results):
    cols = jnp.pad(results.T, ((0, 0), (0, N_PAD - N)), constant_values=-1.0)
    out = _nms_sc(cols.reshape(-1))
    return out.reshape(4, 8)
